# Initial kernel scaffold; baseline (speedup 1.0000x reference)
import jax
import jax.numpy as jnp
from jax.experimental import pallas as pl


def kernel(xs, w):
    raise NotImplementedError("write your pallas kernel here")


# speedup vs baseline: 148.1315x; 148.1315x over previous
import functools

import jax
import jax.numpy as jnp
from jax import lax
from jax.experimental import pallas as pl
from jax.experimental.pallas import tpu as pltpu
from jax.experimental.pallas import tpu_sc as plsc

_BR = 256
_BC = 128
_BLK = _BR * _BC


def _tc_body(Km1, w_ref, lt_ref, contrib_ref, scr):
    b = pl.program_id(0)

    @pl.when(b == 0)
    def _init():
        scr[0] = 0.0
        scr[1] = 0.0

    wv = w_ref[...]
    r = lax.broadcasted_iota(jnp.int32, (_BR, _BC), 0)
    c = lax.broadcasted_iota(jnp.int32, (_BR, _BC), 1)
    k = (b * _BR + r) * _BC + c
    valid = k < Km1
    km = jnp.where(valid, k, 0)
    xo = wv - jnp.log((Km1 - km).astype(jnp.float32))
    l1pe = jnp.log1p(jnp.exp(-jnp.abs(xo)))
    log_z = jnp.minimum(xo, 0.0) - l1pe
    log_1mz = jnp.minimum(-xo, 0.0) - l1pe
    xm = jnp.where(valid, log_1mz, 0.0)
    s = jnp.where(valid, log_z, 0.0)

    tm = lax.broadcasted_iota(jnp.int32, (_BC, _BC), 0)
    tl = lax.broadcasted_iota(jnp.int32, (_BC, _BC), 1)
    t_strict = (tm < tl).astype(jnp.float32)
    rowexcl = lax.dot(xm, t_strict, precision=lax.Precision.HIGHEST)
    rt = rowexcl[:, _BC - 1:_BC] + xm[:, _BC - 1:_BC]
    rtb = jnp.broadcast_to(rt, (_BR, _BC))
    lr = lax.broadcasted_iota(jnp.int32, (_BR, _BR), 0)
    lc = lax.broadcasted_iota(jnp.int32, (_BR, _BR), 1)
    l_strict = (lc < lr).astype(jnp.float32)
    pref = lax.dot(l_strict, rtb, precision=lax.Precision.HIGHEST)

    carry = scr[0]
    cum_prev = carry + rowexcl + pref
    lt_ref[...] = s + cum_prev

    wgt = (Km1 - 1 - km).astype(jnp.float32)
    terms = jnp.where(valid, -xo + 2.0 * log_z, 0.0) + wgt * xm
    scr[1] = scr[1] + jnp.sum(terms)
    scr[0] = carry + jnp.sum(xm)

    @pl.when(b == pl.num_programs(0) - 1)
    def _fin():
        contrib_ref[...] = jnp.full((1, 1), scr[1], jnp.float32)


def _tc_transform(w2d, Km1, grid):
    body = functools.partial(_tc_body, Km1)
    return pl.pallas_call(
        body,
        grid=(grid,),
        in_specs=[pl.BlockSpec((_BR, _BC), lambda b: (b, 0))],
        out_specs=[
            pl.BlockSpec((_BR, _BC), lambda b: (b, 0)),
            pl.BlockSpec((1, 1), lambda b: (0, 0)),
        ],
        out_shape=[
            jax.ShapeDtypeStruct(w2d.shape, jnp.float32),
            jax.ShapeDtypeStruct((1, 1), jnp.float32),
        ],
        scratch_shapes=[pltpu.SMEM((2,), jnp.float32)],
    )(w2d)


def _make_sc_gather(n_idx, n_workers, chunk, n_chunks):
    mesh = plsc.VectorSubcoreMesh(core_axis_name="c", subcore_axis_name="s")
    per_w = n_idx // n_workers

    @functools.partial(
        pl.kernel,
        out_type=jax.ShapeDtypeStruct((n_workers, 16), jnp.float32),
        mesh=mesh,
        scratch_types=[
            pltpu.VMEM((chunk,), jnp.int32),
            pltpu.VMEM((chunk,), jnp.float32),
            pltpu.VMEM((16,), jnp.float32),
            pltpu.SemaphoreType.DMA,
        ],
    )
    def sc_gather(table_hbm, xs_hbm, out_hbm, idx_v, data_v, acc_v, sem):
        wid = lax.axis_index("s") * 2 + lax.axis_index("c")
        base = wid * per_w
        accs = (jnp.zeros((16,), jnp.float32),) * 4

        def chunk_sum(i, a):
            a0, a1, a2, a3 = a
            o = i * 64
            a0 = a0 + data_v[pl.ds(o, 16)]
            a1 = a1 + data_v[pl.ds(o + 16, 16)]
            a2 = a2 + data_v[pl.ds(o + 32, 16)]
            a3 = a3 + data_v[pl.ds(o + 48, 16)]
            return (a0, a1, a2, a3)

        for j in range(n_chunks):
            pltpu.sync_copy(xs_hbm.at[pl.ds(base + j * chunk, chunk)], idx_v)
            pltpu.async_copy(table_hbm.at[idx_v], data_v, sem).wait()
            accs = lax.fori_loop(0, chunk // 64, chunk_sum, accs)

        acc_v[...] = accs[0] + accs[1] + accs[2] + accs[3]
        pltpu.sync_copy(acc_v, out_hbm.at[wid])

    return sc_gather


def kernel(xs, w):
    Km1 = w.shape[0]
    pad = (-Km1) % _BLK
    n_pad = Km1 + pad
    grid = n_pad // _BLK
    w2d = jnp.pad(w, (0, pad)).reshape(n_pad // _BC, _BC)

    lt2d, contrib = _tc_transform(w2d, Km1, grid)
    table = lt2d.reshape(-1)

    xs_flat = xs.reshape(-1)
    n_idx = xs_flat.shape[0]
    n_workers = 32
    per_w = n_idx // n_workers
    chunk = 25600
    n_chunks = per_w // chunk

    partials = _make_sc_gather(n_idx, n_workers, chunk, n_chunks)(
        table, xs_flat)
    return jnp.sum(partials) + contrib[0, 0]


# speedup vs baseline: 154.3547x; 1.0420x over previous
import functools

import jax
import jax.numpy as jnp
from jax import lax
from jax.experimental import pallas as pl
from jax.experimental.pallas import tpu as pltpu
from jax.experimental.pallas import tpu_sc as plsc

_BR = 256
_BC = 128
_BLK = _BR * _BC


def _split_dot(a, b_bf16):
    a_hi = a.astype(jnp.bfloat16)
    a_lo = (a - a_hi.astype(jnp.float32)).astype(jnp.bfloat16)
    return (lax.dot(a_hi, b_bf16, preferred_element_type=jnp.float32)
            + lax.dot(a_lo, b_bf16, preferred_element_type=jnp.float32))


def _split_dot_r(a_bf16, b):
    b_hi = b.astype(jnp.bfloat16)
    b_lo = (b - b_hi.astype(jnp.float32)).astype(jnp.bfloat16)
    return (lax.dot(a_bf16, b_hi, preferred_element_type=jnp.float32)
            + lax.dot(a_bf16, b_lo, preferred_element_type=jnp.float32))


def _tc_body(Km1, w_ref, lt_ref, contrib_ref, scr):
    b = pl.program_id(0)

    @pl.when(b == 0)
    def _init():
        scr[0] = 0.0
        scr[1] = 0.0

    wv = w_ref[...]
    r = lax.broadcasted_iota(jnp.int32, (_BR, _BC), 0)
    c = lax.broadcasted_iota(jnp.int32, (_BR, _BC), 1)
    k = (b * _BR + r) * _BC + c
    valid = k < Km1
    km = jnp.where(valid, k, 0)
    xo = wv - jnp.log((Km1 - km).astype(jnp.float32))
    l1pe = jnp.log1p(jnp.exp(-jnp.abs(xo)))
    log_z = jnp.minimum(xo, 0.0) - l1pe
    log_1mz = jnp.minimum(-xo, 0.0) - l1pe
    xm = jnp.where(valid, log_1mz, 0.0)
    s = jnp.where(valid, log_z, 0.0)

    tm = lax.broadcasted_iota(jnp.int32, (_BC, _BC), 0)
    tl = lax.broadcasted_iota(jnp.int32, (_BC, _BC), 1)
    t_strict = (tm < tl).astype(jnp.bfloat16)
    rowexcl = _split_dot(xm, t_strict)
    rt = rowexcl[:, _BC - 1:_BC] + xm[:, _BC - 1:_BC]
    rtb = jnp.broadcast_to(rt, (_BR, _BC))
    lr = lax.broadcasted_iota(jnp.int32, (_BR, _BR), 0)
    lc = lax.broadcasted_iota(jnp.int32, (_BR, _BR), 1)
    l_strict = (lc < lr).astype(jnp.bfloat16)
    pref = _split_dot_r(l_strict, rtb)

    carry = scr[0]
    cum_prev = carry + rowexcl + pref
    lt_ref[...] = s + cum_prev

    wgt = (Km1 - 1 - km).astype(jnp.float32)
    terms = jnp.where(valid, -xo + 2.0 * log_z, 0.0) + wgt * xm
    scr[1] = scr[1] + jnp.sum(terms)
    scr[0] = carry + jnp.sum(xm)

    @pl.when(b == pl.num_programs(0) - 1)
    def _fin():
        contrib_ref[...] = jnp.full((1, 1), scr[1], jnp.float32)


def _tc_transform(w2d, Km1, grid):
    body = functools.partial(_tc_body, Km1)
    return pl.pallas_call(
        body,
        grid=(grid,),
        in_specs=[pl.BlockSpec((_BR, _BC), lambda b: (b, 0))],
        out_specs=[
            pl.BlockSpec((_BR, _BC), lambda b: (b, 0)),
            pl.BlockSpec((1, 1), lambda b: (0, 0)),
        ],
        out_shape=[
            jax.ShapeDtypeStruct(w2d.shape, jnp.float32),
            jax.ShapeDtypeStruct((1, 1), jnp.float32),
        ],
        scratch_shapes=[pltpu.SMEM((2,), jnp.float32)],
    )(w2d)


def _make_sc_gather(n_idx, n_workers, chunk, n_chunks):
    mesh = plsc.VectorSubcoreMesh(core_axis_name="c", subcore_axis_name="s")
    per_w = n_idx // n_workers

    @functools.partial(
        pl.kernel,
        out_type=jax.ShapeDtypeStruct((n_workers, 16), jnp.float32),
        mesh=mesh,
        scratch_types=[
            pltpu.VMEM((chunk,), jnp.int32),
            pltpu.VMEM((chunk,), jnp.int32),
            pltpu.VMEM((chunk,), jnp.float32),
            pltpu.VMEM((chunk,), jnp.float32),
            pltpu.VMEM((16,), jnp.float32),
            pltpu.SemaphoreType.DMA,
            pltpu.SemaphoreType.DMA,
            pltpu.SemaphoreType.DMA,
            pltpu.SemaphoreType.DMA,
        ],
    )
    def sc_gather(table_hbm, xs_hbm, out_hbm, idx0, idx1, dat0, dat1,
                  acc_v, sem_i0, sem_i1, sem_g0, sem_g1):
        wid = lax.axis_index("s") * 2 + lax.axis_index("c")
        base = wid * per_w
        idxs, dats = (idx0, idx1), (dat0, dat1)
        sem_i, sem_g = (sem_i0, sem_i1), (sem_g0, sem_g1)

        def stage_idx(j, b):
            return pltpu.async_copy(
                xs_hbm.at[pl.ds(base + j * chunk, chunk)], idxs[b], sem_i[b])

        def start_gather(b):
            return pltpu.async_copy(table_hbm.at[idxs[b]], dats[b], sem_g[b])

        def make_sum(b):
            def chunk_sum(i, a):
                a0, a1, a2, a3 = a
                o = i * 64
                a0 = a0 + dats[b][pl.ds(o, 16)]
                a1 = a1 + dats[b][pl.ds(o + 16, 16)]
                a2 = a2 + dats[b][pl.ds(o + 32, 16)]
                a3 = a3 + dats[b][pl.ds(o + 48, 16)]
                return (a0, a1, a2, a3)
            return chunk_sum

        stage_idx(0, 0).wait()
        g = [start_gather(0), None]
        i_h = [None, stage_idx(1, 1) if n_chunks > 1 else None]

        accs = (jnp.zeros((16,), jnp.float32),) * 4
        for j in range(n_chunks):
            b, nb = j & 1, (j + 1) & 1
            if j + 1 < n_chunks:
                i_h[nb].wait()
                g[nb] = start_gather(nb)
            g[b].wait()
            if j + 2 < n_chunks:
                i_h[b] = stage_idx(j + 2, b)
            accs = lax.fori_loop(0, chunk // 64, make_sum(b), accs)

        acc_v[...] = accs[0] + accs[1] + accs[2] + accs[3]
        pltpu.sync_copy(acc_v, out_hbm.at[wid])

    return sc_gather


def kernel(xs, w):
    Km1 = w.shape[0]
    pad = (-Km1) % _BLK
    n_pad = Km1 + pad
    grid = n_pad // _BLK
    w2d = jnp.pad(w, (0, pad)).reshape(n_pad // _BC, _BC)

    lt2d, contrib = _tc_transform(w2d, Km1, grid)
    table = lt2d.reshape(-1)

    xs_flat = xs.reshape(-1)
    n_idx = xs_flat.shape[0]
    n_workers = 32
    per_w = n_idx // n_workers
    chunk = 25600
    n_chunks = per_w // chunk

    partials = _make_sc_gather(n_idx, n_workers, chunk, n_chunks)(
        table, xs_flat)
    return jnp.sum(partials) + contrib[0, 0]


# speedup vs baseline: 154.4174x; 1.0004x over previous
import functools

import jax
import jax.numpy as jnp
from jax import lax
from jax.experimental import pallas as pl
from jax.experimental.pallas import tpu as pltpu
from jax.experimental.pallas import tpu_sc as plsc

_BR = 256
_BC = 128
_BLK = _BR * _BC


def _split_dot(a, b_bf16):
    a_hi = a.astype(jnp.bfloat16)
    a_lo = (a - a_hi.astype(jnp.float32)).astype(jnp.bfloat16)
    return (lax.dot(a_hi, b_bf16, preferred_element_type=jnp.float32)
            + lax.dot(a_lo, b_bf16, preferred_element_type=jnp.float32))


def _split_dot_r(a_bf16, b):
    b_hi = b.astype(jnp.bfloat16)
    b_lo = (b - b_hi.astype(jnp.float32)).astype(jnp.bfloat16)
    return (lax.dot(a_bf16, b_hi, preferred_element_type=jnp.float32)
            + lax.dot(a_bf16, b_lo, preferred_element_type=jnp.float32))


def _tc_body(Km1, w_ref, lt_ref, contrib_ref, scr):
    b = pl.program_id(0)

    @pl.when(b == 0)
    def _init():
        scr[0] = 0.0
        scr[1] = 0.0

    wv = w_ref[...]
    r = lax.broadcasted_iota(jnp.int32, (_BR, _BC), 0)
    c = lax.broadcasted_iota(jnp.int32, (_BR, _BC), 1)
    k = (b * _BR + r) * _BC + c
    valid = k < Km1
    km = jnp.where(valid, k, 0)
    xo = wv - jnp.log((Km1 - km).astype(jnp.float32))
    l1pe = jnp.log1p(jnp.exp(-jnp.abs(xo)))
    log_z = jnp.minimum(xo, 0.0) - l1pe
    log_1mz = jnp.minimum(-xo, 0.0) - l1pe
    xm = jnp.where(valid, log_1mz, 0.0)
    s = jnp.where(valid, log_z, 0.0)

    tm = lax.broadcasted_iota(jnp.int32, (_BC, _BC), 0)
    tl = lax.broadcasted_iota(jnp.int32, (_BC, _BC), 1)
    t_strict = (tm < tl).astype(jnp.bfloat16)
    rowexcl = _split_dot(xm, t_strict)
    rt = rowexcl[:, _BC - 1:_BC] + xm[:, _BC - 1:_BC]
    rtb = jnp.broadcast_to(rt, (_BR, _BC))
    lr = lax.broadcasted_iota(jnp.int32, (_BR, _BR), 0)
    lc = lax.broadcasted_iota(jnp.int32, (_BR, _BR), 1)
    l_strict = (lc < lr).astype(jnp.bfloat16)
    pref = _split_dot_r(l_strict, rtb)

    carry = scr[0]
    cum_prev = carry + rowexcl + pref
    lt_ref[...] = (s + cum_prev).reshape(_BLK)

    wgt = (Km1 - 1 - km).astype(jnp.float32)
    terms = jnp.where(valid, -xo + 2.0 * log_z, 0.0) + wgt * xm
    scr[1] = scr[1] + jnp.sum(terms)
    scr[0] = carry + jnp.sum(xm)

    @pl.when(b == pl.num_programs(0) - 1)
    def _fin():
        contrib_ref[...] = jnp.full((1, 1), scr[1], jnp.float32)


def _tc_transform(w2d, Km1, grid):
    body = functools.partial(_tc_body, Km1)
    return pl.pallas_call(
        body,
        grid=(grid,),
        in_specs=[pl.BlockSpec((_BR, _BC), lambda b: (b, 0))],
        out_specs=[
            pl.BlockSpec((_BLK,), lambda b: (b,)),
            pl.BlockSpec((1, 1), lambda b: (0, 0)),
        ],
        out_shape=[
            jax.ShapeDtypeStruct((w2d.shape[0] * _BC,), jnp.float32),
            jax.ShapeDtypeStruct((1, 1), jnp.float32),
        ],
        scratch_shapes=[pltpu.SMEM((2,), jnp.float32)],
    )(w2d)


def _make_sc_gather(n_idx, n_workers, chunk, n_chunks):
    mesh = plsc.VectorSubcoreMesh(core_axis_name="c", subcore_axis_name="s")
    per_w = n_idx // n_workers

    @functools.partial(
        pl.kernel,
        out_type=jax.ShapeDtypeStruct((n_workers, 16), jnp.float32),
        mesh=mesh,
        scratch_types=[
            pltpu.VMEM((chunk,), jnp.int32),
            pltpu.VMEM((chunk,), jnp.int32),
            pltpu.VMEM((chunk,), jnp.float32),
            pltpu.VMEM((chunk,), jnp.float32),
            pltpu.VMEM((16,), jnp.float32),
            pltpu.SemaphoreType.DMA,
            pltpu.SemaphoreType.DMA,
            pltpu.SemaphoreType.DMA,
            pltpu.SemaphoreType.DMA,
        ],
    )
    def sc_gather(table_hbm, xs_hbm, out_hbm, idx0, idx1, dat0, dat1,
                  acc_v, sem_i0, sem_i1, sem_g0, sem_g1):
        wid = lax.axis_index("s") * 2 + lax.axis_index("c")
        base = wid * per_w
        idxs, dats = (idx0, idx1), (dat0, dat1)
        sem_i, sem_g = (sem_i0, sem_i1), (sem_g0, sem_g1)

        def stage_idx(j, b):
            return pltpu.async_copy(
                xs_hbm.at[pl.ds(base + j * chunk, chunk)], idxs[b], sem_i[b])

        def start_gather(b):
            return pltpu.async_copy(table_hbm.at[idxs[b]], dats[b], sem_g[b])

        def make_sum(b):
            def chunk_sum(i, a):
                a0, a1, a2, a3 = a
                o = i * 64
                a0 = a0 + dats[b][pl.ds(o, 16)]
                a1 = a1 + dats[b][pl.ds(o + 16, 16)]
                a2 = a2 + dats[b][pl.ds(o + 32, 16)]
                a3 = a3 + dats[b][pl.ds(o + 48, 16)]
                return (a0, a1, a2, a3)
            return chunk_sum

        stage_idx(0, 0).wait()
        g = [start_gather(0), None]
        i_h = [None, stage_idx(1, 1) if n_chunks > 1 else None]

        accs = (jnp.zeros((16,), jnp.float32),) * 4
        for j in range(n_chunks):
            b, nb = j & 1, (j + 1) & 1
            if j + 1 < n_chunks:
                i_h[nb].wait()
                g[nb] = start_gather(nb)
            g[b].wait()
            if j + 2 < n_chunks:
                i_h[b] = stage_idx(j + 2, b)
            accs = lax.fori_loop(0, chunk // 64, make_sum(b), accs)

        acc_v[...] = accs[0] + accs[1] + accs[2] + accs[3]
        pltpu.sync_copy(acc_v, out_hbm.at[wid])

    return sc_gather


def kernel(xs, w):
    Km1 = w.shape[0]
    pad = (-Km1) % _BLK
    n_pad = Km1 + pad
    grid = n_pad // _BLK
    w2d = jnp.pad(w, (0, pad)).reshape(n_pad // _BC, _BC)

    table, contrib = _tc_transform(w2d, Km1, grid)

    xs_flat = xs.reshape(-1)
    n_idx = xs_flat.shape[0]
    n_workers = 32
    per_w = n_idx // n_workers
    chunk = 25600
    n_chunks = per_w // chunk

    partials = _make_sc_gather(n_idx, n_workers, chunk, n_chunks)(
        table, xs_flat)
    return jnp.sum(partials) + contrib[0, 0]


# speedup vs baseline: 154.4657x; 1.0003x over previous
import functools

import jax
import jax.numpy as jnp
from jax import lax
from jax.experimental import pallas as pl
from jax.experimental.pallas import tpu as pltpu
from jax.experimental.pallas import tpu_sc as plsc

_BR = 256
_BC = 128
_BLK = _BR * _BC


def _split_dot(a, b_bf16):
    a_hi = a.astype(jnp.bfloat16)
    a_lo = (a - a_hi.astype(jnp.float32)).astype(jnp.bfloat16)
    return (lax.dot(a_hi, b_bf16, preferred_element_type=jnp.float32)
            + lax.dot(a_lo, b_bf16, preferred_element_type=jnp.float32))


def _split_dot_r(a_bf16, b):
    b_hi = b.astype(jnp.bfloat16)
    b_lo = (b - b_hi.astype(jnp.float32)).astype(jnp.bfloat16)
    return (lax.dot(a_bf16, b_hi, preferred_element_type=jnp.float32)
            + lax.dot(a_bf16, b_lo, preferred_element_type=jnp.float32))


def _tc_body(Km1, w_ref, lt_ref, contrib_ref, scr):
    b = pl.program_id(0)

    @pl.when(b == 0)
    def _init():
        scr[0] = 0.0
        scr[1] = 0.0

    wv = w_ref[...]
    r = lax.broadcasted_iota(jnp.int32, (_BR, _BC), 0)
    c = lax.broadcasted_iota(jnp.int32, (_BR, _BC), 1)
    k = (b * _BR + r) * _BC + c
    valid = k < Km1
    km = jnp.where(valid, k, 0)
    xo = wv - jnp.log((Km1 - km).astype(jnp.float32))
    l1pe = jnp.log1p(jnp.exp(-jnp.abs(xo)))
    log_z = jnp.minimum(xo, 0.0) - l1pe
    log_1mz = jnp.minimum(-xo, 0.0) - l1pe
    xm = jnp.where(valid, log_1mz, 0.0)
    s = jnp.where(valid, log_z, 0.0)

    tm = lax.broadcasted_iota(jnp.int32, (_BC, _BC), 0)
    tl = lax.broadcasted_iota(jnp.int32, (_BC, _BC), 1)
    t_strict = (tm < tl).astype(jnp.bfloat16)
    rowexcl = _split_dot(xm, t_strict)
    rt = rowexcl[:, _BC - 1:_BC] + xm[:, _BC - 1:_BC]
    rtb = jnp.broadcast_to(rt, (_BR, _BC))
    lr = lax.broadcasted_iota(jnp.int32, (_BR, _BR), 0)
    lc = lax.broadcasted_iota(jnp.int32, (_BR, _BR), 1)
    l_strict = (lc < lr).astype(jnp.bfloat16)
    pref = _split_dot_r(l_strict, rtb)

    carry = scr[0]
    cum_prev = carry + rowexcl + pref
    lt_ref[...] = (s + cum_prev).reshape(_BLK)

    wgt = (Km1 - 1 - km).astype(jnp.float32)
    terms = jnp.where(valid, -xo + 2.0 * log_z, 0.0) + wgt * xm
    scr[1] = scr[1] + jnp.sum(terms)
    scr[0] = carry + jnp.sum(xm)

    @pl.when(b == pl.num_programs(0) - 1)
    def _fin():
        contrib_ref[...] = jnp.full((1, 1), scr[1], jnp.float32)


def _tc_transform(w2d, Km1, grid):
    body = functools.partial(_tc_body, Km1)
    return pl.pallas_call(
        body,
        grid=(grid,),
        in_specs=[pl.BlockSpec((_BR, _BC), lambda b: (b, 0))],
        out_specs=[
            pl.BlockSpec((_BLK,), lambda b: (b,)),
            pl.BlockSpec((1, 1), lambda b: (0, 0)),
        ],
        out_shape=[
            jax.ShapeDtypeStruct((w2d.shape[0] * _BC,), jnp.float32),
            jax.ShapeDtypeStruct((1, 1), jnp.float32),
        ],
        scratch_shapes=[pltpu.SMEM((2,), jnp.float32)],
    )(w2d)


def _make_sc_gather(n_idx, n_workers, chunk, n_chunks):
    mesh = plsc.VectorSubcoreMesh(core_axis_name="c", subcore_axis_name="s")
    per_w = n_idx // n_workers

    @functools.partial(
        pl.kernel,
        out_type=jax.ShapeDtypeStruct((n_workers, 16), jnp.float32),
        mesh=mesh,
        compiler_params=pltpu.CompilerParams(use_tc_tiling_on_sc=True),
        scratch_types=[
            pltpu.VMEM((chunk,), jnp.int32),
            pltpu.VMEM((chunk,), jnp.int32),
            pltpu.VMEM((chunk,), jnp.float32),
            pltpu.VMEM((chunk,), jnp.float32),
            pltpu.VMEM((16,), jnp.float32),
            pltpu.SemaphoreType.DMA,
            pltpu.SemaphoreType.DMA,
            pltpu.SemaphoreType.DMA,
            pltpu.SemaphoreType.DMA,
        ],
    )
    def sc_gather(table_hbm, xs_hbm, out_hbm, idx0, idx1, dat0, dat1,
                  acc_v, sem_i0, sem_i1, sem_g0, sem_g1):
        wid = lax.axis_index("s") * 2 + lax.axis_index("c")
        base = wid * per_w
        idxs, dats = (idx0, idx1), (dat0, dat1)
        sem_i, sem_g = (sem_i0, sem_i1), (sem_g0, sem_g1)

        def stage_idx(j, b):
            return pltpu.async_copy(
                xs_hbm.at[pl.ds(base + j * chunk, chunk)], idxs[b], sem_i[b])

        def start_gather(b):
            return pltpu.async_copy(table_hbm.at[idxs[b]], dats[b], sem_g[b])

        def make_sum(b):
            def chunk_sum(i, a):
                a0, a1, a2, a3 = a
                o = i * 64
                a0 = a0 + dats[b][pl.ds(o, 16)]
                a1 = a1 + dats[b][pl.ds(o + 16, 16)]
                a2 = a2 + dats[b][pl.ds(o + 32, 16)]
                a3 = a3 + dats[b][pl.ds(o + 48, 16)]
                return (a0, a1, a2, a3)
            return chunk_sum

        stage_idx(0, 0).wait()
        g = [start_gather(0), None]
        i_h = [None, stage_idx(1, 1) if n_chunks > 1 else None]

        accs = (jnp.zeros((16,), jnp.float32),) * 4
        for j in range(n_chunks):
            b, nb = j & 1, (j + 1) & 1
            if j + 1 < n_chunks:
                i_h[nb].wait()
                g[nb] = start_gather(nb)
            g[b].wait()
            if j + 2 < n_chunks:
                i_h[b] = stage_idx(j + 2, b)
            accs = lax.fori_loop(0, chunk // 64, make_sum(b), accs)

        acc_v[...] = accs[0] + accs[1] + accs[2] + accs[3]
        pltpu.sync_copy(acc_v, out_hbm.at[wid])

    return sc_gather


def kernel(xs, w):
    Km1 = w.shape[0]
    pad = (-Km1) % _BLK
    n_pad = Km1 + pad
    grid = n_pad // _BLK
    w2d = jnp.pad(w, (0, pad)).reshape(n_pad // _BC, _BC)

    table, contrib = _tc_transform(w2d, Km1, grid)

    xs_flat = xs.reshape(-1)
    n_idx = xs_flat.shape[0]
    n_workers = 32
    per_w = n_idx // n_workers
    chunk = 25600
    n_chunks = per_w // chunk

    partials = _make_sc_gather(n_idx, n_workers, chunk, n_chunks)(
        table, xs_flat)
    return jnp.sum(partials) + contrib[0, 0]


# speedup vs baseline: 173.3499x; 1.1223x over previous
import functools

import jax
import jax.numpy as jnp
from jax import lax
from jax.experimental import pallas as pl
from jax.experimental.pallas import tpu as pltpu
from jax.experimental.pallas import tpu_sc as plsc

_BR = 256
_BC = 128
_BLK = _BR * _BC


def _split_dot(a, b_bf16):
    a_hi = a.astype(jnp.bfloat16)
    a_lo = (a - a_hi.astype(jnp.float32)).astype(jnp.bfloat16)
    return (lax.dot(a_hi, b_bf16, preferred_element_type=jnp.float32)
            + lax.dot(a_lo, b_bf16, preferred_element_type=jnp.float32))


def _split_dot_r(a_bf16, b):
    b_hi = b.astype(jnp.bfloat16)
    b_lo = (b - b_hi.astype(jnp.float32)).astype(jnp.bfloat16)
    return (lax.dot(a_bf16, b_hi, preferred_element_type=jnp.float32)
            + lax.dot(a_bf16, b_lo, preferred_element_type=jnp.float32))


def _tc_body(Km1, n_xs_blk, w_ref, xs_ref, lt_ref, xsf_ref, contrib_ref, scr):
    xsf_ref[...] = xs_ref[...].reshape(n_xs_blk)
    b = pl.program_id(0)

    @pl.when(b == 0)
    def _init():
        scr[0] = 0.0
        scr[1] = 0.0

    wv = w_ref[...]
    r = lax.broadcasted_iota(jnp.int32, (_BR, _BC), 0)
    c = lax.broadcasted_iota(jnp.int32, (_BR, _BC), 1)
    k = (b * _BR + r) * _BC + c
    valid = k < Km1
    km = jnp.where(valid, k, 0)
    xo = wv - jnp.log((Km1 - km).astype(jnp.float32))
    l1pe = jnp.log1p(jnp.exp(-jnp.abs(xo)))
    log_z = jnp.minimum(xo, 0.0) - l1pe
    log_1mz = jnp.minimum(-xo, 0.0) - l1pe
    xm = jnp.where(valid, log_1mz, 0.0)
    s = jnp.where(valid, log_z, 0.0)

    tm = lax.broadcasted_iota(jnp.int32, (_BC, _BC), 0)
    tl = lax.broadcasted_iota(jnp.int32, (_BC, _BC), 1)
    t_strict = (tm < tl).astype(jnp.bfloat16)
    rowexcl = _split_dot(xm, t_strict)
    rt = rowexcl[:, _BC - 1:_BC] + xm[:, _BC - 1:_BC]
    rtb = jnp.broadcast_to(rt, (_BR, _BC))
    lr = lax.broadcasted_iota(jnp.int32, (_BR, _BR), 0)
    lc = lax.broadcasted_iota(jnp.int32, (_BR, _BR), 1)
    l_strict = (lc < lr).astype(jnp.bfloat16)
    pref = _split_dot_r(l_strict, rtb)

    carry = scr[0]
    cum_prev = carry + rowexcl + pref
    lt_ref[...] = (s + cum_prev).reshape(_BLK)

    wgt = (Km1 - 1 - km).astype(jnp.float32)
    terms = jnp.where(valid, -xo + 2.0 * log_z, 0.0) + wgt * xm
    scr[1] = scr[1] + jnp.sum(terms)
    scr[0] = carry + jnp.sum(xm)

    @pl.when(b == pl.num_programs(0) - 1)
    def _fin():
        contrib_ref[...] = jnp.full((1, 1), scr[1], jnp.float32)


def _tc_transform(w2d, xs_t, Km1, grid):
    n_xs = xs_t.shape[0] * xs_t.shape[1]
    xs_cols = xs_t.shape[1] // grid
    n_xs_blk = xs_t.shape[0] * xs_cols
    body = functools.partial(_tc_body, Km1, n_xs_blk)
    return pl.pallas_call(
        body,
        grid=(grid,),
        in_specs=[
            pl.BlockSpec((_BR, _BC), lambda b: (b, 0)),
            pl.BlockSpec((xs_t.shape[0], xs_cols), lambda b: (0, b)),
        ],
        out_specs=[
            pl.BlockSpec((_BLK,), lambda b: (b,)),
            pl.BlockSpec((n_xs_blk,), lambda b: (b,)),
            pl.BlockSpec((1, 1), lambda b: (0, 0)),
        ],
        out_shape=[
            jax.ShapeDtypeStruct((w2d.shape[0] * _BC,), jnp.float32),
            jax.ShapeDtypeStruct((n_xs,), jnp.int32),
            jax.ShapeDtypeStruct((1, 1), jnp.float32),
        ],
        scratch_shapes=[pltpu.SMEM((2,), jnp.float32)],
    )(w2d, xs_t)


def _make_sc_gather(n_idx, n_workers, chunk, n_chunks):
    mesh = plsc.VectorSubcoreMesh(core_axis_name="c", subcore_axis_name="s")
    per_w = n_idx // n_workers

    @functools.partial(
        pl.kernel,
        out_type=jax.ShapeDtypeStruct((n_workers, 16), jnp.float32),
        mesh=mesh,
        compiler_params=pltpu.CompilerParams(use_tc_tiling_on_sc=True),
        scratch_types=[
            pltpu.VMEM((chunk,), jnp.int32),
            pltpu.VMEM((chunk,), jnp.int32),
            pltpu.VMEM((chunk,), jnp.float32),
            pltpu.VMEM((chunk,), jnp.float32),
            pltpu.VMEM((16,), jnp.float32),
            pltpu.SemaphoreType.DMA,
            pltpu.SemaphoreType.DMA,
            pltpu.SemaphoreType.DMA,
            pltpu.SemaphoreType.DMA,
        ],
    )
    def sc_gather(table_hbm, xs_hbm, out_hbm, idx0, idx1, dat0, dat1,
                  acc_v, sem_i0, sem_i1, sem_g0, sem_g1):
        wid = lax.axis_index("s") * 2 + lax.axis_index("c")
        base = wid * per_w
        idxs, dats = (idx0, idx1), (dat0, dat1)
        sem_i, sem_g = (sem_i0, sem_i1), (sem_g0, sem_g1)

        def stage_idx(j, b):
            return pltpu.async_copy(
                xs_hbm.at[pl.ds(base + j * chunk, chunk)], idxs[b], sem_i[b])

        def start_gather(b):
            return pltpu.async_copy(table_hbm.at[idxs[b]], dats[b], sem_g[b])

        def make_sum(b):
            def chunk_sum(i, a):
                a0, a1, a2, a3 = a
                o = i * 64
                a0 = a0 + dats[b][pl.ds(o, 16)]
                a1 = a1 + dats[b][pl.ds(o + 16, 16)]
                a2 = a2 + dats[b][pl.ds(o + 32, 16)]
                a3 = a3 + dats[b][pl.ds(o + 48, 16)]
                return (a0, a1, a2, a3)
            return chunk_sum

        stage_idx(0, 0).wait()
        g = [start_gather(0), None]
        i_h = [None, stage_idx(1, 1) if n_chunks > 1 else None]

        accs = (jnp.zeros((16,), jnp.float32),) * 4
        for j in range(n_chunks):
            b, nb = j & 1, (j + 1) & 1
            if j + 1 < n_chunks:
                i_h[nb].wait()
                g[nb] = start_gather(nb)
            g[b].wait()
            if j + 2 < n_chunks:
                i_h[b] = stage_idx(j + 2, b)
            accs = lax.fori_loop(0, chunk // 64, make_sum(b), accs)

        acc_v[...] = accs[0] + accs[1] + accs[2] + accs[3]
        pltpu.sync_copy(acc_v, out_hbm.at[wid])

    return sc_gather


def kernel(xs, w):
    Km1 = w.shape[0]
    grid = 32
    n_pad = grid * _BLK
    w2d = jnp.pad(w, (0, n_pad - Km1)).reshape(n_pad // _BC, _BC)
    xs_t = xs.T

    table, xs_flat, contrib = _tc_transform(w2d, xs_t, Km1, grid)

    n_idx = xs.size
    n_workers = 32
    per_w = n_idx // n_workers
    chunk = 25600
    n_chunks = per_w // chunk

    partials = _make_sc_gather(n_idx, n_workers, chunk, n_chunks)(
        table, xs_flat)
    return jnp.sum(partials) + contrib[0, 0]


# speedup vs baseline: 176.0917x; 1.0158x over previous
import functools

import jax
import jax.numpy as jnp
import numpy as np
from jax import lax
from jax.experimental import pallas as pl
from jax.experimental.pallas import tpu as pltpu
from jax.experimental.pallas import tpu_sc as plsc

_BR = 248
_BC = 128
_BLK = _BR * _BC
_GRID = 32


def _split_dot(a, b_bf16):
    a_hi = a.astype(jnp.bfloat16)
    a_lo = (a - a_hi.astype(jnp.float32)).astype(jnp.bfloat16)
    return (lax.dot(a_hi, b_bf16, preferred_element_type=jnp.float32)
            + lax.dot(a_lo, b_bf16, preferred_element_type=jnp.float32))


def _split_dot_r(a_bf16, b):
    b_hi = b.astype(jnp.bfloat16)
    b_lo = (b - b_hi.astype(jnp.float32)).astype(jnp.bfloat16)
    return (lax.dot(a_bf16, b_hi, preferred_element_type=jnp.float32)
            + lax.dot(a_bf16, b_lo, preferred_element_type=jnp.float32))


def _tc_body(Km1, n_xs_blk, w_ref, loff_ref, xs_ref, lt_ref, xsf_ref,
             contrib_ref, scr):
    xsf_ref[...] = xs_ref[...].reshape(n_xs_blk)
    b = pl.program_id(0)

    @pl.when(b == 0)
    def _init():
        scr[0] = 0.0
        scr[1] = 0.0

    wv = w_ref[...].reshape(_BR, _BC)
    r = lax.broadcasted_iota(jnp.int32, (_BR, _BC), 0)
    c = lax.broadcasted_iota(jnp.int32, (_BR, _BC), 1)
    k = (b * _BR + r) * _BC + c
    valid = k < Km1
    km = jnp.where(valid, k, 0)
    xo = wv - loff_ref[...]
    l1pe = jnp.log1p(jnp.exp(-jnp.abs(xo)))
    log_z = jnp.minimum(xo, 0.0) - l1pe
    log_1mz = jnp.minimum(-xo, 0.0) - l1pe
    xm = jnp.where(valid, log_1mz, 0.0)
    s = jnp.where(valid, log_z, 0.0)

    tm = lax.broadcasted_iota(jnp.int32, (_BC, _BC), 0)
    tl = lax.broadcasted_iota(jnp.int32, (_BC, _BC), 1)
    t_strict = (tm < tl).astype(jnp.bfloat16)
    rowexcl = _split_dot(xm, t_strict)
    rt = rowexcl[:, _BC - 1:_BC] + xm[:, _BC - 1:_BC]
    rtb = jnp.broadcast_to(rt, (_BR, _BC))
    lr = lax.broadcasted_iota(jnp.int32, (_BR, _BR), 0)
    lc = lax.broadcasted_iota(jnp.int32, (_BR, _BR), 1)
    l_strict = (lc < lr).astype(jnp.bfloat16)
    pref = _split_dot_r(l_strict, rtb)

    carry = scr[0]
    cum_prev = carry + rowexcl + pref
    lt_ref[...] = (s + cum_prev).reshape(_BLK)

    wgt = (Km1 - 1 - km).astype(jnp.float32)
    terms = jnp.where(valid, -xo + 2.0 * log_z, 0.0) + wgt * xm
    scr[1] = scr[1] + jnp.sum(terms)
    scr[0] = carry + jnp.sum(xm)

    @pl.when(b == pl.num_programs(0) - 1)
    def _fin():
        contrib_ref[...] = jnp.full((1, 1), scr[1], jnp.float32)


def _tc_transform(w, loff2d, xs_t, Km1, grid):
    n_xs = xs_t.shape[0] * xs_t.shape[1]
    xs_cols = xs_t.shape[1] // grid
    n_xs_blk = xs_t.shape[0] * xs_cols
    body = functools.partial(_tc_body, Km1, n_xs_blk)
    return pl.pallas_call(
        body,
        grid=(grid,),
        in_specs=[
            pl.BlockSpec((_BLK,), lambda b: (b,)),
            pl.BlockSpec((_BR, _BC), lambda b: (b, 0)),
            pl.BlockSpec((xs_t.shape[0], xs_cols), lambda b: (0, b)),
        ],
        out_specs=[
            pl.BlockSpec((_BLK,), lambda b: (b,)),
            pl.BlockSpec((n_xs_blk,), lambda b: (b,)),
            pl.BlockSpec((1, 1), lambda b: (0, 0)),
        ],
        out_shape=[
            jax.ShapeDtypeStruct((grid * _BLK,), jnp.float32),
            jax.ShapeDtypeStruct((n_xs,), jnp.int32),
            jax.ShapeDtypeStruct((1, 1), jnp.float32),
        ],
        scratch_shapes=[pltpu.SMEM((2,), jnp.float32)],
    )(w, loff2d, xs_t)


def _make_sc_gather(n_idx, n_workers, chunk, n_chunks):
    mesh = plsc.VectorSubcoreMesh(core_axis_name="c", subcore_axis_name="s")
    per_w = n_idx // n_workers

    @functools.partial(
        pl.kernel,
        out_type=jax.ShapeDtypeStruct((n_workers, 16), jnp.float32),
        mesh=mesh,
        compiler_params=pltpu.CompilerParams(use_tc_tiling_on_sc=True),
        scratch_types=[
            pltpu.VMEM((chunk,), jnp.int32),
            pltpu.VMEM((chunk,), jnp.int32),
            pltpu.VMEM((chunk,), jnp.float32),
            pltpu.VMEM((chunk,), jnp.float32),
            pltpu.VMEM((16,), jnp.float32),
            pltpu.SemaphoreType.DMA,
            pltpu.SemaphoreType.DMA,
            pltpu.SemaphoreType.DMA,
            pltpu.SemaphoreType.DMA,
        ],
    )
    def sc_gather(table_hbm, xs_hbm, out_hbm, idx0, idx1, dat0, dat1,
                  acc_v, sem_i0, sem_i1, sem_g0, sem_g1):
        wid = lax.axis_index("s") * 2 + lax.axis_index("c")
        base = wid * per_w
        idxs, dats = (idx0, idx1), (dat0, dat1)
        sem_i, sem_g = (sem_i0, sem_i1), (sem_g0, sem_g1)

        def stage_idx(j, b):
            return pltpu.async_copy(
                xs_hbm.at[pl.ds(base + j * chunk, chunk)], idxs[b], sem_i[b])

        def start_gather(b):
            return pltpu.async_copy(table_hbm.at[idxs[b]], dats[b], sem_g[b])

        def make_sum(b):
            def chunk_sum(i, a):
                a0, a1, a2, a3 = a
                o = i * 64
                a0 = a0 + dats[b][pl.ds(o, 16)]
                a1 = a1 + dats[b][pl.ds(o + 16, 16)]
                a2 = a2 + dats[b][pl.ds(o + 32, 16)]
                a3 = a3 + dats[b][pl.ds(o + 48, 16)]
                return (a0, a1, a2, a3)
            return chunk_sum

        stage_idx(0, 0).wait()
        g = [start_gather(0), None]
        i_h = [None, stage_idx(1, 1) if n_chunks > 1 else None]

        accs = (jnp.zeros((16,), jnp.float32),) * 4
        for j in range(n_chunks):
            b, nb = j & 1, (j + 1) & 1
            if j + 1 < n_chunks:
                i_h[nb].wait()
                g[nb] = start_gather(nb)
            g[b].wait()
            if j + 2 < n_chunks:
                i_h[b] = stage_idx(j + 2, b)
            accs = lax.fori_loop(0, chunk // 64, make_sum(b), accs)

        acc_v[...] = accs[0] + accs[1] + accs[2] + accs[3]
        pltpu.sync_copy(acc_v, out_hbm.at[wid])

    return sc_gather


def kernel(xs, w):
    Km1 = w.shape[0]
    grid = _GRID
    n_pad = grid * _BLK
    loff_np = np.zeros((n_pad,), np.float32)
    loff_np[:Km1] = np.log(np.arange(Km1, 0, -1, dtype=np.float32))
    loff2d = jnp.asarray(loff_np.reshape(n_pad // _BC, _BC))
    xs_t = xs.T

    table, xs_flat, contrib = _tc_transform(w, loff2d, xs_t, Km1, grid)

    n_idx = xs.size
    n_workers = 32
    per_w = n_idx // n_workers
    chunk = 25600
    n_chunks = per_w // chunk

    partials = _make_sc_gather(n_idx, n_workers, chunk, n_chunks)(
        table, xs_flat)
    return jnp.sum(partials) + contrib[0, 0]


# speedup vs baseline: 181.9259x; 1.0331x over previous
import functools

import jax
import jax.numpy as jnp
import numpy as np
from jax import lax
from jax.experimental import pallas as pl
from jax.experimental.pallas import tpu as pltpu
from jax.experimental.pallas import tpu_sc as plsc

_BR = 496
_BC = 128
_BLK = _BR * _BC
_GRID = 16


def _split_dot(a, b_bf16):
    a_hi = a.astype(jnp.bfloat16)
    a_lo = (a - a_hi.astype(jnp.float32)).astype(jnp.bfloat16)
    return (lax.dot(a_hi, b_bf16, preferred_element_type=jnp.float32)
            + lax.dot(a_lo, b_bf16, preferred_element_type=jnp.float32))


def _split_dot_r(a_bf16, b):
    b_hi = b.astype(jnp.bfloat16)
    b_lo = (b - b_hi.astype(jnp.float32)).astype(jnp.bfloat16)
    return (lax.dot(a_bf16, b_hi, preferred_element_type=jnp.float32)
            + lax.dot(a_bf16, b_lo, preferred_element_type=jnp.float32))


def _tc_body(Km1, n_xs_blk, w_ref, loff_ref, xs_ref, lt_ref, xsf_ref,
             contrib_ref, scr):
    xsf_ref[...] = xs_ref[...].reshape(n_xs_blk)
    b = pl.program_id(0)

    @pl.when(b == 0)
    def _init():
        scr[0] = 0.0
        scr[1] = 0.0

    wv = w_ref[...].reshape(_BR, _BC)
    r = lax.broadcasted_iota(jnp.int32, (_BR, _BC), 0)
    c = lax.broadcasted_iota(jnp.int32, (_BR, _BC), 1)
    k = (b * _BR + r) * _BC + c
    valid = k < Km1
    km = jnp.where(valid, k, 0)
    xo = wv - loff_ref[...]
    l1pe = jnp.log1p(jnp.exp(-jnp.abs(xo)))
    log_z = jnp.minimum(xo, 0.0) - l1pe
    log_1mz = jnp.minimum(-xo, 0.0) - l1pe
    xm = jnp.where(valid, log_1mz, 0.0)
    s = jnp.where(valid, log_z, 0.0)

    tm = lax.broadcasted_iota(jnp.int32, (_BC, _BC), 0)
    tl = lax.broadcasted_iota(jnp.int32, (_BC, _BC), 1)
    t_strict = (tm < tl).astype(jnp.bfloat16)
    rowexcl = _split_dot(xm, t_strict)
    rt = rowexcl[:, _BC - 1:_BC] + xm[:, _BC - 1:_BC]
    lr = lax.broadcasted_iota(jnp.int32, (_BR, _BR), 0)
    lc = lax.broadcasted_iota(jnp.int32, (_BR, _BR), 1)
    l_strict = (lc < lr).astype(jnp.bfloat16)
    pref = jnp.broadcast_to(_split_dot_r(l_strict, rt), (_BR, _BC))

    carry = scr[0]
    cum_prev = carry + rowexcl + pref
    lt_ref[...] = (s + cum_prev).reshape(_BLK)

    wgt = (Km1 - 1 - km).astype(jnp.float32)
    terms = jnp.where(valid, -xo + 2.0 * log_z, 0.0) + wgt * xm
    scr[1] = scr[1] + jnp.sum(terms)
    scr[0] = carry + jnp.sum(xm)

    @pl.when(b == pl.num_programs(0) - 1)
    def _fin():
        contrib_ref[...] = jnp.full((1, 1), scr[1], jnp.float32)


def _tc_transform(w, loff2d, xs_t, Km1, grid):
    n_xs = xs_t.shape[0] * xs_t.shape[1]
    xs_cols = xs_t.shape[1] // grid
    n_xs_blk = xs_t.shape[0] * xs_cols
    body = functools.partial(_tc_body, Km1, n_xs_blk)
    return pl.pallas_call(
        body,
        grid=(grid,),
        in_specs=[
            pl.BlockSpec((_BLK,), lambda b: (b,)),
            pl.BlockSpec((_BR, _BC), lambda b: (b, 0)),
            pl.BlockSpec((xs_t.shape[0], xs_cols), lambda b: (0, b)),
        ],
        out_specs=[
            pl.BlockSpec((_BLK,), lambda b: (b,)),
            pl.BlockSpec((n_xs_blk,), lambda b: (b,)),
            pl.BlockSpec((1, 1), lambda b: (0, 0)),
        ],
        out_shape=[
            jax.ShapeDtypeStruct((grid * _BLK,), jnp.float32),
            jax.ShapeDtypeStruct((n_xs,), jnp.int32),
            jax.ShapeDtypeStruct((1, 1), jnp.float32),
        ],
        scratch_shapes=[pltpu.SMEM((2,), jnp.float32)],
    )(w, loff2d, xs_t)


def _make_sc_gather(n_idx, n_workers, chunk, n_chunks):
    mesh = plsc.VectorSubcoreMesh(core_axis_name="c", subcore_axis_name="s")
    per_w = n_idx // n_workers

    @functools.partial(
        pl.kernel,
        out_type=jax.ShapeDtypeStruct((n_workers, 16), jnp.float32),
        mesh=mesh,
        compiler_params=pltpu.CompilerParams(use_tc_tiling_on_sc=True),
        scratch_types=[
            pltpu.VMEM((chunk,), jnp.int32),
            pltpu.VMEM((chunk,), jnp.int32),
            pltpu.VMEM((chunk,), jnp.float32),
            pltpu.VMEM((chunk,), jnp.float32),
            pltpu.VMEM((16,), jnp.float32),
            pltpu.SemaphoreType.DMA,
            pltpu.SemaphoreType.DMA,
            pltpu.SemaphoreType.DMA,
            pltpu.SemaphoreType.DMA,
        ],
    )
    def sc_gather(table_hbm, xs_hbm, out_hbm, idx0, idx1, dat0, dat1,
                  acc_v, sem_i0, sem_i1, sem_g0, sem_g1):
        wid = lax.axis_index("s") * 2 + lax.axis_index("c")
        base = wid * per_w
        idxs, dats = (idx0, idx1), (dat0, dat1)
        sem_i, sem_g = (sem_i0, sem_i1), (sem_g0, sem_g1)

        def stage_idx(j, b):
            return pltpu.async_copy(
                xs_hbm.at[pl.ds(base + j * chunk, chunk)], idxs[b], sem_i[b])

        def start_gather(b):
            return pltpu.async_copy(table_hbm.at[idxs[b]], dats[b], sem_g[b])

        def make_sum(b):
            def chunk_sum(i, a):
                a0, a1, a2, a3 = a
                o = i * 64
                a0 = a0 + dats[b][pl.ds(o, 16)]
                a1 = a1 + dats[b][pl.ds(o + 16, 16)]
                a2 = a2 + dats[b][pl.ds(o + 32, 16)]
                a3 = a3 + dats[b][pl.ds(o + 48, 16)]
                return (a0, a1, a2, a3)
            return chunk_sum

        stage_idx(0, 0).wait()
        g = [start_gather(0), None]
        i_h = [None, stage_idx(1, 1) if n_chunks > 1 else None]

        accs = (jnp.zeros((16,), jnp.float32),) * 4
        for j in range(n_chunks):
            b, nb = j & 1, (j + 1) & 1
            if j + 1 < n_chunks:
                i_h[nb].wait()
                g[nb] = start_gather(nb)
            g[b].wait()
            if j + 2 < n_chunks:
                i_h[b] = stage_idx(j + 2, b)
            accs = lax.fori_loop(0, chunk // 64, make_sum(b), accs)

        acc_v[...] = accs[0] + accs[1] + accs[2] + accs[3]
        pltpu.sync_copy(acc_v, out_hbm.at[wid])

    return sc_gather


def kernel(xs, w):
    Km1 = w.shape[0]
    grid = _GRID
    n_pad = grid * _BLK
    loff_np = np.zeros((n_pad,), np.float32)
    loff_np[:Km1] = np.log(np.arange(Km1, 0, -1, dtype=np.float32))
    loff2d = jnp.asarray(loff_np.reshape(n_pad // _BC, _BC))
    xs_t = xs.T

    table, xs_flat, contrib = _tc_transform(w, loff2d, xs_t, Km1, grid)

    n_idx = xs.size
    n_workers = 32
    per_w = n_idx // n_workers
    chunk = 25600
    n_chunks = per_w // chunk

    partials = _make_sc_gather(n_idx, n_workers, chunk, n_chunks)(
        table, xs_flat)
    return jnp.sum(partials) + contrib[0, 0]


# speedup vs baseline: 183.3899x; 1.0080x over previous
import functools

import jax
import jax.numpy as jnp
import numpy as np
from jax import lax
from jax.experimental import pallas as pl
from jax.experimental.pallas import tpu as pltpu
from jax.experimental.pallas import tpu_sc as plsc

_BR = 496
_BC = 128
_BLK = _BR * _BC
_GRID = 16


def _split_dot(a, b_bf16):
    a_hi = a.astype(jnp.bfloat16)
    a_lo = (a - a_hi.astype(jnp.float32)).astype(jnp.bfloat16)
    return (lax.dot(a_hi, b_bf16, preferred_element_type=jnp.float32)
            + lax.dot(a_lo, b_bf16, preferred_element_type=jnp.float32))


def _split_dot_r(a_bf16, b):
    b_hi = b.astype(jnp.bfloat16)
    b_lo = (b - b_hi.astype(jnp.float32)).astype(jnp.bfloat16)
    return (lax.dot(a_bf16, b_hi, preferred_element_type=jnp.float32)
            + lax.dot(a_bf16, b_lo, preferred_element_type=jnp.float32))


def _tc_body(Km1, n_xs_blk, w_ref, loff_ref, xs_ref, lt_ref, xsf_ref,
             contrib_ref, scr):
    xsf_ref[...] = xs_ref[...].reshape(n_xs_blk)
    b = pl.program_id(0)

    @pl.when(b == 0)
    def _init():
        scr[0] = 0.0
        scr[1] = 0.0

    wv = w_ref[...].reshape(_BR, _BC)
    r = lax.broadcasted_iota(jnp.int32, (_BR, _BC), 0)
    c = lax.broadcasted_iota(jnp.int32, (_BR, _BC), 1)
    k = (b * _BR + r) * _BC + c
    valid = k < Km1
    km = jnp.where(valid, k, 0)
    xo = wv - loff_ref[...]
    l1pe = jnp.log1p(jnp.exp(-jnp.abs(xo)))
    log_z = jnp.minimum(xo, 0.0) - l1pe
    log_1mz = jnp.minimum(-xo, 0.0) - l1pe
    xm = jnp.where(valid, log_1mz, 0.0)
    s = jnp.where(valid, log_z, 0.0)

    tm = lax.broadcasted_iota(jnp.int32, (_BC, _BC), 0)
    tl = lax.broadcasted_iota(jnp.int32, (_BC, _BC), 1)
    t_strict = (tm < tl).astype(jnp.bfloat16)
    rowexcl = _split_dot(xm, t_strict)
    rt = rowexcl[:, _BC - 1:_BC] + xm[:, _BC - 1:_BC]
    lr = lax.broadcasted_iota(jnp.int32, (_BR, _BR), 0)
    lc = lax.broadcasted_iota(jnp.int32, (_BR, _BR), 1)
    l_strict = (lc < lr).astype(jnp.bfloat16)
    pref = jnp.broadcast_to(_split_dot_r(l_strict, rt), (_BR, _BC))

    carry = scr[0]
    cum_prev = carry + rowexcl + pref
    lt_ref[...] = (s + cum_prev).reshape(_BLK)

    wgt = (Km1 - 1 - km).astype(jnp.float32)
    terms = jnp.where(valid, -xo + 2.0 * log_z, 0.0) + wgt * xm
    scr[1] = scr[1] + jnp.sum(terms)
    scr[0] = carry + jnp.sum(xm)

    @pl.when(b == pl.num_programs(0) - 1)
    def _fin():
        contrib_ref[...] = jnp.full((1, 1), scr[1], jnp.float32)


def _tc_transform(w, loff2d, xs_t, Km1, grid):
    n_xs = xs_t.shape[0] * xs_t.shape[1]
    xs_cols = xs_t.shape[1] // grid
    n_xs_blk = xs_t.shape[0] * xs_cols
    body = functools.partial(_tc_body, Km1, n_xs_blk)
    return pl.pallas_call(
        body,
        grid=(grid,),
        in_specs=[
            pl.BlockSpec((_BLK,), lambda b: (b,)),
            pl.BlockSpec((_BR, _BC), lambda b: (b, 0)),
            pl.BlockSpec((xs_t.shape[0], xs_cols), lambda b: (0, b)),
        ],
        out_specs=[
            pl.BlockSpec((_BLK,), lambda b: (b,)),
            pl.BlockSpec((n_xs_blk,), lambda b: (b,)),
            pl.BlockSpec((1, 1), lambda b: (0, 0)),
        ],
        out_shape=[
            jax.ShapeDtypeStruct((grid * _BLK,), jnp.float32),
            jax.ShapeDtypeStruct((n_xs,), jnp.int32),
            jax.ShapeDtypeStruct((1, 1), jnp.float32),
        ],
        scratch_shapes=[pltpu.SMEM((2,), jnp.float32)],
    )(w, loff2d, xs_t)


def _make_sc_gather(n_idx, n_workers, chunk, n_chunks):
    mesh = plsc.VectorSubcoreMesh(core_axis_name="c", subcore_axis_name="s")
    per_w = n_idx // n_workers

    @functools.partial(
        pl.kernel,
        out_type=jax.ShapeDtypeStruct((n_workers, 16), jnp.float32),
        mesh=mesh,
        compiler_params=pltpu.CompilerParams(use_tc_tiling_on_sc=True),
        scratch_types=[
            pltpu.VMEM((chunk,), jnp.int32),
            pltpu.VMEM((chunk,), jnp.int32),
            pltpu.VMEM((chunk,), jnp.float32),
            pltpu.VMEM((chunk,), jnp.float32),
            pltpu.VMEM((16,), jnp.float32),
            pltpu.SemaphoreType.DMA,
            pltpu.SemaphoreType.DMA,
            pltpu.SemaphoreType.DMA,
            pltpu.SemaphoreType.DMA,
        ],
    )
    def sc_gather(table_hbm, xs_hbm, out_hbm, idx0, idx1, dat0, dat1,
                  acc_v, sem_i0, sem_i1, sem_g0, sem_g1):
        wid = lax.axis_index("s") * 2 + lax.axis_index("c")
        base = wid * per_w
        idxs, dats = (idx0, idx1), (dat0, dat1)
        sem_i, sem_g = (sem_i0, sem_i1), (sem_g0, sem_g1)

        def stage_idx(j, b):
            return pltpu.async_copy(
                xs_hbm.at[pl.ds(base + j * chunk, chunk)], idxs[b], sem_i[b])

        def start_gather(b):
            return pltpu.async_copy(table_hbm.at[idxs[b]], dats[b], sem_g[b])

        def make_sum(b):
            def chunk_sum(i, a):
                a0, a1, a2, a3 = a
                o = i * 64
                a0 = a0 + dats[b][pl.ds(o, 16)]
                a1 = a1 + dats[b][pl.ds(o + 16, 16)]
                a2 = a2 + dats[b][pl.ds(o + 32, 16)]
                a3 = a3 + dats[b][pl.ds(o + 48, 16)]
                return (a0, a1, a2, a3)
            return chunk_sum

        stage_idx(0, 0).wait()
        g = [start_gather(0), None]
        i_h = [None, stage_idx(1, 1) if n_chunks > 1 else None]

        accs = (jnp.zeros((16,), jnp.float32),) * 4
        for j in range(n_chunks):
            b, nb = j & 1, (j + 1) & 1
            if j + 1 < n_chunks:
                i_h[nb].wait()
                g[nb] = start_gather(nb)
            g[b].wait()
            if j + 2 < n_chunks:
                i_h[b] = stage_idx(j + 2, b)
            accs = lax.fori_loop(0, chunk // 64, make_sum(b), accs)

        acc_v[...] = accs[0] + accs[1] + accs[2] + accs[3]
        pltpu.sync_copy(acc_v, out_hbm.at[wid])

    return sc_gather


def kernel(xs, w):
    Km1 = w.shape[0]
    grid = _GRID
    n_pad = grid * _BLK
    loff_np = np.zeros((n_pad,), np.float32)
    loff_np[:Km1] = np.log(np.arange(Km1, 0, -1, dtype=np.float32))
    loff2d = jnp.asarray(loff_np.reshape(n_pad // _BC, _BC))
    xs_t = xs.T

    table, xs_flat, contrib = _tc_transform(w, loff2d, xs_t, Km1, grid)

    n_idx = xs.size
    n_workers = 32
    per_w = n_idx // n_workers
    chunk = 12800
    n_chunks = per_w // chunk

    partials = _make_sc_gather(n_idx, n_workers, chunk, n_chunks)(
        table, xs_flat)
    return jnp.sum(partials) + contrib[0, 0]


# speedup vs baseline: 185.4499x; 1.0112x over previous
import functools

import jax
import jax.numpy as jnp
import numpy as np
from jax import lax
from jax.experimental import pallas as pl
from jax.experimental.pallas import tpu as pltpu
from jax.experimental.pallas import tpu_sc as plsc

_BR = 496
_BC = 128
_BLK = _BR * _BC
_GRID = 16


def _split_dot(a, b_bf16):
    a_hi = a.astype(jnp.bfloat16)
    a_lo = (a - a_hi.astype(jnp.float32)).astype(jnp.bfloat16)
    return (lax.dot(a_hi, b_bf16, preferred_element_type=jnp.float32)
            + lax.dot(a_lo, b_bf16, preferred_element_type=jnp.float32))


def _split_dot_r(a_bf16, b):
    b_hi = b.astype(jnp.bfloat16)
    b_lo = (b - b_hi.astype(jnp.float32)).astype(jnp.bfloat16)
    return (lax.dot(a_bf16, b_hi, preferred_element_type=jnp.float32)
            + lax.dot(a_bf16, b_lo, preferred_element_type=jnp.float32))


def _tc_body(Km1, n_xs_blk, w_ref, loff_ref, xs_ref, lt_ref, xsf_ref,
             contrib_ref, scr):
    xsf_ref[...] = xs_ref[...].reshape(n_xs_blk)
    b = pl.program_id(0)

    @pl.when(b == 0)
    def _init():
        scr[0] = 0.0
        scr[1] = 0.0

    wv = w_ref[...].reshape(_BR, _BC)
    r = lax.broadcasted_iota(jnp.int32, (_BR, _BC), 0)
    c = lax.broadcasted_iota(jnp.int32, (_BR, _BC), 1)
    k = (b * _BR + r) * _BC + c
    valid = k < Km1
    km = jnp.where(valid, k, 0)
    xo = wv - loff_ref[...]
    l1pe = jnp.log1p(jnp.exp(-jnp.abs(xo)))
    log_z = jnp.minimum(xo, 0.0) - l1pe
    log_1mz = jnp.minimum(-xo, 0.0) - l1pe
    xm = jnp.where(valid, log_1mz, 0.0)
    s = jnp.where(valid, log_z, 0.0)

    tm = lax.broadcasted_iota(jnp.int32, (_BC, _BC), 0)
    tl = lax.broadcasted_iota(jnp.int32, (_BC, _BC), 1)
    t_strict = (tm < tl).astype(jnp.bfloat16)
    rowexcl = _split_dot(xm, t_strict)
    rt = rowexcl[:, _BC - 1:_BC] + xm[:, _BC - 1:_BC]
    lr = lax.broadcasted_iota(jnp.int32, (_BR, _BR), 0)
    lc = lax.broadcasted_iota(jnp.int32, (_BR, _BR), 1)
    l_strict = (lc < lr).astype(jnp.bfloat16)
    pref = jnp.broadcast_to(_split_dot_r(l_strict, rt), (_BR, _BC))

    carry = scr[0]
    cum_prev = carry + rowexcl + pref
    lt_ref[...] = (s + cum_prev).reshape(_BLK)

    wgt = (Km1 - 1 - km).astype(jnp.float32)
    terms = jnp.where(valid, -xo + 2.0 * log_z, 0.0) + wgt * xm
    scr[1] = scr[1] + jnp.sum(terms)
    scr[0] = carry + jnp.sum(xm)

    @pl.when(b == pl.num_programs(0) - 1)
    def _fin():
        contrib_ref[...] = jnp.full((1, 1), scr[1], jnp.float32)


def _tc_transform(w, loff2d, xs_t, Km1, grid):
    n_xs = xs_t.shape[0] * xs_t.shape[1]
    xs_cols = xs_t.shape[1] // grid
    n_xs_blk = xs_t.shape[0] * xs_cols
    body = functools.partial(_tc_body, Km1, n_xs_blk)
    return pl.pallas_call(
        body,
        grid=(grid,),
        in_specs=[
            pl.BlockSpec((_BLK,), lambda b: (b,)),
            pl.BlockSpec((_BR, _BC), lambda b: (b, 0)),
            pl.BlockSpec((xs_t.shape[0], xs_cols), lambda b: (0, b)),
        ],
        out_specs=[
            pl.BlockSpec((_BLK,), lambda b: (b,)),
            pl.BlockSpec((n_xs_blk,), lambda b: (b,)),
            pl.BlockSpec((1, 1), lambda b: (0, 0)),
        ],
        out_shape=[
            jax.ShapeDtypeStruct((grid * _BLK,), jnp.float32),
            jax.ShapeDtypeStruct((n_xs,), jnp.int32),
            jax.ShapeDtypeStruct((1, 1), jnp.float32),
        ],
        scratch_shapes=[pltpu.SMEM((2,), jnp.float32)],
    )(w, loff2d, xs_t)


def _make_sc_gather(n_idx, n_workers, chunk, n_chunks):
    mesh = plsc.VectorSubcoreMesh(core_axis_name="c", subcore_axis_name="s")
    per_w = n_idx // n_workers

    @functools.partial(
        pl.kernel,
        out_type=jax.ShapeDtypeStruct((n_workers, 16), jnp.float32),
        mesh=mesh,
        compiler_params=pltpu.CompilerParams(use_tc_tiling_on_sc=True),
        scratch_types=[
            pltpu.VMEM((chunk,), jnp.int32),
            pltpu.VMEM((chunk,), jnp.int32),
            pltpu.VMEM((chunk,), jnp.float32),
            pltpu.VMEM((chunk,), jnp.float32),
            pltpu.VMEM((16,), jnp.float32),
            pltpu.SemaphoreType.DMA,
            pltpu.SemaphoreType.DMA,
            pltpu.SemaphoreType.DMA,
            pltpu.SemaphoreType.DMA,
        ],
    )
    def sc_gather(table_hbm, xs_hbm, out_hbm, idx0, idx1, dat0, dat1,
                  acc_v, sem_i0, sem_i1, sem_g0, sem_g1):
        wid = lax.axis_index("s") * 2 + lax.axis_index("c")
        base = wid * per_w
        idxs, dats = (idx0, idx1), (dat0, dat1)
        sem_i, sem_g = (sem_i0, sem_i1), (sem_g0, sem_g1)

        def stage_idx(j, b):
            return pltpu.async_copy(
                xs_hbm.at[pl.ds(base + j * chunk, chunk)], idxs[b], sem_i[b])

        def start_gather(b):
            return pltpu.async_copy(table_hbm.at[idxs[b]], dats[b], sem_g[b])

        def make_sum(b):
            def chunk_sum(i, a):
                a0, a1, a2, a3 = a
                o = i * 64
                a0 = a0 + dats[b][pl.ds(o, 16)]
                a1 = a1 + dats[b][pl.ds(o + 16, 16)]
                a2 = a2 + dats[b][pl.ds(o + 32, 16)]
                a3 = a3 + dats[b][pl.ds(o + 48, 16)]
                return (a0, a1, a2, a3)
            return chunk_sum

        stage_idx(0, 0).wait()
        g = [start_gather(0), None]
        i_h = [None, stage_idx(1, 1) if n_chunks > 1 else None]

        accs = (jnp.zeros((16,), jnp.float32),) * 4
        for j in range(n_chunks):
            b, nb = j & 1, (j + 1) & 1
            if j + 1 < n_chunks:
                i_h[nb].wait()
                g[nb] = start_gather(nb)
            g[b].wait()
            if j + 2 < n_chunks:
                i_h[b] = stage_idx(j + 2, b)
            accs = lax.fori_loop(0, chunk // 64, make_sum(b), accs)

        acc_v[...] = accs[0] + accs[1] + accs[2] + accs[3]
        pltpu.sync_copy(acc_v, out_hbm.at[wid])

    return sc_gather


def kernel(xs, w):
    Km1 = w.shape[0]
    grid = _GRID
    n_pad = grid * _BLK
    loff_np = np.zeros((n_pad,), np.float32)
    loff_np[:Km1] = np.log(np.arange(Km1, 0, -1, dtype=np.float32))
    loff2d = jnp.asarray(loff_np.reshape(n_pad // _BC, _BC))
    xs_t = xs.T

    table, xs_flat, contrib = _tc_transform(w, loff2d, xs_t, Km1, grid)

    n_idx = xs.size
    n_workers = 32
    per_w = n_idx // n_workers
    chunk = 6400
    n_chunks = per_w // chunk

    partials = _make_sc_gather(n_idx, n_workers, chunk, n_chunks)(
        table, xs_flat)
    return jnp.sum(partials) + contrib[0, 0]
